# baseline (speedup 1.0000x reference)
import functools

import jax
import jax.numpy as jnp
from jax import lax
from jax.experimental import pallas as pl
from jax.experimental.pallas import tpu as pltpu
from jax.experimental.pallas import tpu_sc as plsc

RESO = 32
GRID = RESO ** 3
B = 4
N = 65536
F = 32
CHUNK = 4096
W = 2048

_DENOM = 1 + 0.1 + 10e-4
_CLIP_HI = 1.0 - 10e-4


def _enc_body(p_ref, fw_ref, fb_ref, w0_ref, b0_ref, w1_ref, b1_ref, ws_ref,
              net_ref, idx_ref):
    pblk = p_ref[0]
    p_nor = jnp.clip(pblk / _DENOM + 0.5, 0.0, _CLIP_HI)
    xi = jnp.clip((p_nor * RESO).astype(jnp.int32), 0, RESO - 1)
    idx = xi[0:1, :] + RESO * xi[1:2, :] + (RESO * RESO) * xi[2:3, :]
    idx_ref[0] = idx.astype(jnp.float32)

    x = jnp.dot(fw_ref[...], pblk, preferred_element_type=jnp.float32)
    x = x + fb_ref[...]
    rx = jax.nn.relu(x)
    n1 = jnp.dot(w0_ref[...], rx, preferred_element_type=jnp.float32) + b0_ref[...]
    dx = jnp.dot(w1_ref[...], jax.nn.relu(n1),
                 preferred_element_type=jnp.float32) + b1_ref[...]
    xs = jnp.dot(ws_ref[...], x, preferred_element_type=jnp.float32)
    net_ref[0] = xs + dx


def _encode(p_t, params):
    pr0 = params["blocks"][0]
    full = lambda a: pl.BlockSpec(a.shape, lambda b, i: (0,) * a.ndim)
    fw = params["fc_pos_w"]
    fb = params["fc_pos_b"].reshape(64, 1)
    args = (fw, fb, pr0["w0"], pr0["b0"].reshape(32, 1),
            pr0["w1"], pr0["b1"].reshape(32, 1), pr0["ws"])
    return pl.pallas_call(
        _enc_body,
        grid=(B, N // CHUNK),
        in_specs=[pl.BlockSpec((1, 3, CHUNK), lambda b, i: (b, 0, i))]
        + [full(a) for a in args],
        out_specs=[pl.BlockSpec((1, F, CHUNK), lambda b, i: (b, 0, i)),
                   pl.BlockSpec((1, 1, CHUNK), lambda b, i: (b, 0, i))],
        out_shape=[jax.ShapeDtypeStruct((B, F, N), jnp.float32),
                   jax.ShapeDtypeStruct((B, 1, N), jnp.float32)],
    )(p_t, *args)


def _res_body(has_fc, net_in_ref, pool_ref, w0_ref, b0_ref, w1_ref, b1_ref,
              ws_ref, *rest):
    if has_fc:
        fcw_ref, fcb_ref, out_ref = rest
    else:
        out_ref, = rest
    x = jnp.concatenate([net_in_ref[0], pool_ref[0]], axis=0)
    n1 = jnp.dot(w0_ref[...], jax.nn.relu(x),
                 preferred_element_type=jnp.float32) + b0_ref[...]
    dx = jnp.dot(w1_ref[...], jax.nn.relu(n1),
                 preferred_element_type=jnp.float32) + b1_ref[...]
    xs = jnp.dot(ws_ref[...], x, preferred_element_type=jnp.float32)
    out = xs + dx
    if has_fc:
        out = jnp.dot(fcw_ref[...], out,
                      preferred_element_type=jnp.float32) + fcb_ref[...]
    out_ref[0] = out


def _resnet(net, pooled, pr, fc=None):
    full = lambda a: pl.BlockSpec(a.shape, lambda b, i: (0,) * a.ndim)
    args = [pr["w0"], pr["b0"].reshape(32, 1), pr["w1"],
            pr["b1"].reshape(32, 1), pr["ws"]]
    if fc is not None:
        args += [fc[0], fc[1].reshape(32, 1)]
    blk = pl.BlockSpec((1, F, CHUNK), lambda b, i: (b, 0, i))
    return pl.pallas_call(
        functools.partial(_res_body, fc is not None),
        grid=(B, N // CHUNK),
        in_specs=[blk, blk] + [full(a) for a in args],
        out_specs=blk,
        out_shape=jax.ShapeDtypeStruct((B, F, N), jnp.float32),
    )(net, pooled, *args)


@functools.cache
def _sc_mesh():
    return plsc.VectorSubcoreMesh(core_axis_name="c", subcore_axis_name="s")


def _scatter_max(acc, iv, v):
    cur = plsc.load_gather(acc, [iv])
    pend0 = jnp.where(v > cur, jnp.int32(1), jnp.int32(0))

    def cond(pend_i):
        return jnp.max(pend_i, axis=0) > 0

    def body(pend_i):
        pend = pend_i != 0
        plsc.store_scatter(acc, [iv], v, mask=pend)
        chk = plsc.load_gather(acc, [iv])
        return jnp.where(pend & (v > chk), jnp.int32(1), jnp.int32(0))

    lax.while_loop(cond, body, pend0)


def _fill(ref, n, value):
    vec = jnp.full((16,), value, jnp.float32)

    def bd(j, _):
        ref[pl.ds(j * 16, 16)] = vec
        return 0

    lax.fori_loop(0, n // 16, bd, 0)


def _sc_pool(net, idxf):
    return _sc_pool_kernel()(net, idxf)


@functools.cache
def _sc_pool_kernel():
    return functools.partial(
        pl.kernel, mesh=_sc_mesh(),
        compiler_params=pltpu.CompilerParams(needs_layout_passes=False),
        out_type=jax.ShapeDtypeStruct((B, F, N), jnp.float32),
        scratch_types=[pltpu.VMEM((GRID,), jnp.float32),
                       pltpu.VMEM((GRID,), jnp.float32),
                       pltpu.VMEM((W,), jnp.float32),
                       pltpu.VMEM((W,), jnp.float32),
                       pltpu.VMEM((W,), jnp.float32)])(_sc_pool_body)


def _sc_pool_body(net_hbm, idxf_hbm, out_hbm, acc0, acc1, idxw, v0w, v1w):
    c = lax.axis_index("c")
    s = lax.axis_index("s")
    f0 = 2 * s
    f1 = 2 * s + 1
    for b_loc in range(2):
        b = 2 * c + b_loc
        _fill(acc0, GRID, -jnp.inf)
        _fill(acc1, GRID, -jnp.inf)

        def win_scatter(w, _):
            base = w * W
            pltpu.sync_copy(idxf_hbm.at[b, 0, pl.ds(base, W)], idxw)
            pltpu.sync_copy(net_hbm.at[b, f0, pl.ds(base, W)], v0w)
            pltpu.sync_copy(net_hbm.at[b, f1, pl.ds(base, W)], v1w)

            def vec_body(i, _):
                iv = idxw[pl.ds(i * 16, 16)].astype(jnp.int32)
                _scatter_max(acc0, iv, v0w[pl.ds(i * 16, 16)])
                _scatter_max(acc1, iv, v1w[pl.ds(i * 16, 16)])
                return 0

            lax.fori_loop(0, W // 16, vec_body, 0)
            return 0

        lax.fori_loop(0, N // W, win_scatter, 0)

        def win_gather(w, _):
            base = w * W
            pltpu.sync_copy(idxf_hbm.at[b, 0, pl.ds(base, W)], idxw)

            def vec_body(i, _):
                iv = idxw[pl.ds(i * 16, 16)].astype(jnp.int32)
                v0w[pl.ds(i * 16, 16)] = plsc.load_gather(acc0, [iv])
                v1w[pl.ds(i * 16, 16)] = plsc.load_gather(acc1, [iv])
                return 0

            lax.fori_loop(0, W // 16, vec_body, 0)
            pltpu.sync_copy(v0w, out_hbm.at[b, f0, pl.ds(base, W)])
            pltpu.sync_copy(v1w, out_hbm.at[b, f1, pl.ds(base, W)])
            return 0

        lax.fori_loop(0, N // W, win_gather, 0)


_GDN = lax.GatherDimensionNumbers(
    offset_dims=(), collapsed_slice_dims=(0,), start_index_map=(0,))


def _perm16(x, ind):
    return lax.gather(x, ind[:, None], _GDN, (1,),
                      mode=lax.GatherScatterMode.PROMISE_IN_BOUNDS)


def _sc_mean(net, idxf):
    return _sc_mean_kernel()(net, idxf)


@functools.cache
def _sc_mean_kernel():
    return functools.partial(
        pl.kernel, mesh=_sc_mesh(),
        compiler_params=pltpu.CompilerParams(needs_layout_passes=False),
        out_type=jax.ShapeDtypeStruct((B, F, GRID), jnp.float32),
        scratch_types=[pltpu.VMEM((GRID,), jnp.float32),
                       pltpu.VMEM((GRID,), jnp.float32),
                       pltpu.VMEM((GRID,), jnp.float32),
                       pltpu.VMEM((W,), jnp.float32),
                       pltpu.VMEM((W,), jnp.float32),
                       pltpu.VMEM((W,), jnp.float32)])(_sc_mean_body)


def _sc_mean_body(net_hbm, idxf_hbm, out_hbm, acc0, acc1, cnt, idxw, v0w, v1w):
    c = lax.axis_index("c")
    s = lax.axis_index("s")
    f0 = 2 * s
    f1 = 2 * s + 1
    lane = lax.iota(jnp.int32, 16)
    for b_loc in range(2):
        b = 2 * c + b_loc
        _fill(acc0, GRID, 0.0)
        _fill(acc1, GRID, 0.0)
        _fill(cnt, GRID, 0.0)

        def win_scatter(w, _):
            base = w * W
            pltpu.sync_copy(idxf_hbm.at[b, 0, pl.ds(base, W)], idxw)
            pltpu.sync_copy(net_hbm.at[b, f0, pl.ds(base, W)], v0w)
            pltpu.sync_copy(net_hbm.at[b, f1, pl.ds(base, W)], v1w)

            def vec_body(i, _):
                iv = idxw[pl.ds(i * 16, 16)].astype(jnp.int32)
                k, perm = plsc.sort_key_val(iv, lane)
                pv0 = _perm16(v0w[pl.ds(i * 16, 16)], perm)
                pv1 = _perm16(v1w[pl.ds(i * 16, 16)], perm)
                ct = jnp.full((16,), 1.0, jnp.float32)
                for sft in (1, 2, 4, 8):
                    src = jnp.maximum(lane - sft, 0)
                    same = (_perm16(k, src) == k) & (lane >= sft)
                    pv0 = pv0 + jnp.where(same, _perm16(pv0, src), 0.0)
                    pv1 = pv1 + jnp.where(same, _perm16(pv1, src), 0.0)
                    ct = ct + jnp.where(same, _perm16(ct, src), 0.0)
                nxt = jnp.minimum(lane + 1, 15)
                last = (lane == 15) | (_perm16(k, nxt) != k)
                plsc.addupdate_scatter(acc0, [k], pv0, mask=last)
                plsc.addupdate_scatter(acc1, [k], pv1, mask=last)
                plsc.addupdate_scatter(cnt, [k], ct, mask=last)
                return 0

            lax.fori_loop(0, W // 16, vec_body, 0)
            return 0

        lax.fori_loop(0, N // W, win_scatter, 0)

        def win_out(w, _):
            base = w * W

            def vec_body(i, _):
                sl = pl.ds(base + i * 16, 16)
                d = jnp.maximum(cnt[sl], 1.0)
                v0w[pl.ds(i * 16, 16)] = acc0[sl] / d
                v1w[pl.ds(i * 16, 16)] = acc1[sl] / d
                return 0

            lax.fori_loop(0, W // 16, vec_body, 0)
            pltpu.sync_copy(v0w, out_hbm.at[b, f0, pl.ds(base, W)])
            pltpu.sync_copy(v1w, out_hbm.at[b, f1, pl.ds(base, W)])
            return 0

        lax.fori_loop(0, GRID // W, win_out, 0)


def _group_norm(x, gamma, beta, groups=8, eps=1e-5):
    Bb, C = x.shape[0], x.shape[1]
    xr = x.reshape(Bb, groups, C // groups, -1)
    m = xr.mean(axis=(2, 3), keepdims=True)
    v = xr.var(axis=(2, 3), keepdims=True)
    xr = (xr - m) / jnp.sqrt(v + eps)
    x = xr.reshape(x.shape)
    return x * gamma[None, :, None, None, None] + beta[None, :, None, None, None]


def _conv3d(x, w, b, pad=1):
    out = lax.conv_general_dilated(
        x, w, (1, 1, 1), [(pad, pad)] * 3,
        dimension_numbers=("NCDHW", "OIDHW", "NCDHW"))
    return out + b[None, :, None, None, None]


def _single_conv(x, g, bb, w, b):
    return jax.nn.relu(_conv3d(_group_norm(x, g, bb), w, b))


def _double_conv(x, pr):
    x = _single_conv(x, pr["gn1_g"], pr["gn1_b"], pr["w1"], pr["b1"])
    x = _single_conv(x, pr["gn2_g"], pr["gn2_b"], pr["w2"], pr["b2"])
    return x


def _maxpool3d(x):
    return lax.reduce_window(x, -jnp.inf, lax.max,
                             (1, 1, 2, 2, 2), (1, 1, 2, 2, 2), "VALID")


def _upsample2(x):
    x = jnp.repeat(x, 2, axis=2)
    x = jnp.repeat(x, 2, axis=3)
    return jnp.repeat(x, 2, axis=4)


def _unet3d(x, pr):
    e0 = _double_conv(x, pr["enc0"])
    e1 = _double_conv(_maxpool3d(e0), pr["enc1"])
    e2 = _double_conv(_maxpool3d(e1), pr["enc2"])
    d1 = _double_conv(jnp.concatenate([e1, _upsample2(e2)], axis=1), pr["dec1"])
    d0 = _double_conv(jnp.concatenate([e0, _upsample2(d1)], axis=1), pr["dec0"])
    return _conv3d(d0, pr["final_w"], pr["final_b"], pad=0)


def kernel(p, params):
    p_t = p.transpose(0, 2, 1)
    net, idxf = _encode(p_t, params)
    for i, blk in enumerate(params["blocks"][1:]):
        pooled = _sc_pool(net, idxf)
        fc = ((params["fc_c_w"], params["fc_c_b"]) if i == 3 else None)
        net = _resnet(net, pooled, blk, fc=fc)
    grid_rows = _sc_mean(net, idxf)
    grid = grid_rows.reshape(B, F, RESO, RESO, RESO)
    return _unet3d(grid, params["unet"])
